# baseline (speedup 1.0000x reference)
import functools

import jax
import jax.numpy as jnp
from jax import lax
from jax.experimental import pallas as pl
from jax.experimental.pallas import tpu as pltpu
from jax.experimental.pallas import tpu_sc as plsc

N_NODES = 10000
N_EDGES = 160000
IN_CH = 256
HID_CH = 256
OUT_CH = 128

NC = 2
NS = 16
L = 16
NW = NC * NS

HP = 10240
DUMP = N_NODES
EC = 128
EP = 163840

RB = 512


def _sc_mesh():
    return plsc.VectorSubcoreMesh(core_axis_name="c", subcore_axis_name="s")


def _make_deg():
    epw = EP // NW
    nchunks = epw // EC

    @functools.partial(
        pl.kernel,
        out_type=jax.ShapeDtypeStruct((NW, HP), jnp.float32),
        mesh=_sc_mesh(),
        scratch_types=[
            pltpu.VMEM((EC,), jnp.int32),
            pltpu.VMEM((HP,), jnp.float32),
        ],
        compiler_params=pltpu.CompilerParams(needs_layout_passes=False),
    )
    def deg_kernel(dst_hbm, deg_hbm, dstv, hist):
        c = lax.axis_index("c")
        s = lax.axis_index("s")
        wid = c * NS + s
        zeros = jnp.zeros((L,), jnp.float32)

        def zero_body(i, carry):
            hist[pl.ds(i * L, L)] = zeros
            return carry

        lax.fori_loop(0, HP // L, zero_body, 0)

        ones = jnp.ones((L,), jnp.float32)

        def edge_body(k, carry):
            base = wid * epw + k * EC
            pltpu.sync_copy(dst_hbm.at[pl.ds(base, EC)], dstv)
            for j in range(EC // L):
                idx = dstv[pl.ds(j * L, L)]
                plsc.addupdate_scatter(hist, [idx], ones)
            return carry

        lax.fori_loop(0, nchunks, edge_body, 0)
        pltpu.sync_copy(hist, deg_hbm.at[wid])

    return deg_kernel


def _edge_loop(h, epairs, acc, idxb, rows, sem0, sem1, chunk0, nchunks):
    def load_idx(ch, b):
        pltpu.sync_copy(epairs.at[ch], idxb.at[b])

    def start_gather(b, sem):
        pltpu.async_copy(h.at[idxb.at[b, 0]], rows.at[b], sem)

    def wait_gather(b, sem):
        pltpu.make_async_copy(h.at[idxb.at[b, 0]], rows.at[b], sem).wait()

    def scatter(b):
        pltpu.sync_copy(rows.at[b], acc.at[idxb.at[b, 1]], add=True)

    load_idx(chunk0, 0)
    start_gather(0, sem0)
    nh = nchunks // 2

    def body(m, carry):
        ch = chunk0 + 2 * m
        wait_gather(0, sem0)
        load_idx(ch + 1, 1)
        start_gather(1, sem1)
        scatter(0)
        wait_gather(1, sem1)

        @pl.when(m < nh - 1)
        def _():
            load_idx(ch + 2, 0)
            start_gather(0, sem0)

        scatter(1)
        return carry

    lax.fori_loop(0, nh, body, 0)


def _agg_scratch(F):
    return [
        pltpu.VMEM((2, 2, EC), jnp.int32),
        pltpu.VMEM((2, EC, F), jnp.float32),
        pltpu.VMEM_SHARED((HP, F), jnp.float32),
        pltpu.SemaphoreType.DMA,
        pltpu.SemaphoreType.DMA,
    ]


def _make_agg(F):
    rpw = HP // NS
    nchunks = EP // EC // NS

    @functools.partial(
        pl.kernel,
        out_type=(jax.ShapeDtypeStruct((HP, F), jnp.float32),
                  jax.ShapeDtypeStruct((HP, F), jnp.float32)),
        mesh=_sc_mesh(),
        scratch_types=_agg_scratch(F),
    )
    def agg_kernel(h0, h1, zrows, epairs, o0, o1, idxb, rows, acc,
                   sem0, sem1):
        c = lax.axis_index("c")
        s = lax.axis_index("s")
        r0 = s * rpw
        chunk0 = s * nchunks

        pltpu.sync_copy(zrows, acc.at[pl.ds(r0, rpw)])
        plsc.subcore_barrier()

        @pl.when(c == 0)
        def _():
            _edge_loop(h0, epairs, acc, idxb, rows, sem0, sem1,
                       chunk0, nchunks)

        @pl.when(c == 1)
        def _():
            _edge_loop(h1, epairs, acc, idxb, rows, sem0, sem1,
                       chunk0, nchunks)

        plsc.subcore_barrier()

        def writeback(o):
            pltpu.sync_copy(acc.at[pl.ds(r0, rpw)], o.at[pl.ds(r0, rpw)])

        @pl.when(c == 0)
        def _():
            writeback(o0)

        @pl.when(c == 1)
        def _():
            writeback(o1)

    return agg_kernel


def _make_agg_edge_split(F):
    rpw = HP // NS
    nchunks = EP // EC // NW

    @functools.partial(
        pl.kernel,
        out_type=(jax.ShapeDtypeStruct((HP, F), jnp.float32),
                  jax.ShapeDtypeStruct((HP, F), jnp.float32)),
        mesh=_sc_mesh(),
        scratch_types=_agg_scratch(F),
    )
    def agg_kernel(g, zrows, epairs, o0, o1, idxb, rows, acc, sem0, sem1):
        c = lax.axis_index("c")
        s = lax.axis_index("s")
        wid = c * NS + s
        r0 = s * rpw

        pltpu.sync_copy(zrows, acc.at[pl.ds(r0, rpw)])
        plsc.subcore_barrier()
        _edge_loop(g, epairs, acc, idxb, rows, sem0, sem1,
                   wid * nchunks, nchunks)
        plsc.subcore_barrier()

        @pl.when(c == 0)
        def _():
            pltpu.sync_copy(acc.at[pl.ds(r0, rpw)], o0.at[pl.ds(r0, rpw)])

        @pl.when(c == 1)
        def _():
            pltpu.sync_copy(acc.at[pl.ds(r0, rpw)], o1.at[pl.ds(r0, rpw)])

    return agg_kernel


def _dinv_of(deg_blk):
    return lax.rsqrt(1.0 + jnp.sum(deg_blk, axis=0))


def _mm1_body(x_ref, w_ref, deg_ref, h0_ref, h1_ref):
    dinv = _dinv_of(deg_ref[...])
    h = jnp.dot(x_ref[...], w_ref[...], preferred_element_type=jnp.float32)
    h = h * dinv[:, None]
    h0_ref[...] = h[:, :HID_CH // 2]
    h1_ref[...] = h[:, HID_CH // 2:]


def _mm1(x_p, W1, deg_parts):
    grid = (HP // RB,)
    return pl.pallas_call(
        _mm1_body,
        grid=grid,
        in_specs=[
            pl.BlockSpec((RB, IN_CH), lambda i: (i, 0)),
            pl.BlockSpec((IN_CH, HID_CH), lambda i: (0, 0)),
            pl.BlockSpec((NW, RB), lambda i: (0, i)),
        ],
        out_specs=[
            pl.BlockSpec((RB, HID_CH // 2), lambda i: (i, 0)),
            pl.BlockSpec((RB, HID_CH // 2), lambda i: (i, 0)),
        ],
        out_shape=[
            jax.ShapeDtypeStruct((HP, HID_CH // 2), jnp.float32),
            jax.ShapeDtypeStruct((HP, HID_CH // 2), jnp.float32),
        ],
    )(x_p, W1, deg_parts)


def _mm2_body(a0_ref, a1_ref, h0_ref, h1_ref, deg_ref, b1_ref, w2_ref,
              g_ref):
    dinv = _dinv_of(deg_ref[...])
    b = b1_ref[...]
    t0 = a0_ref[...] + h0_ref[...]
    t1 = a1_ref[...] + h1_ref[...]
    z0 = jnp.maximum(t0 * dinv[:, None] + b[:, :HID_CH // 2], 0.0)
    z1 = jnp.maximum(t1 * dinv[:, None] + b[:, HID_CH // 2:], 0.0)
    w2 = w2_ref[...]
    h = jnp.dot(z0, w2[:HID_CH // 2], preferred_element_type=jnp.float32)
    h = h + jnp.dot(z1, w2[HID_CH // 2:], preferred_element_type=jnp.float32)
    g_ref[...] = h * dinv[:, None]


def _mm2(a0, a1, h0, h1, deg_parts, b1r, W2):
    grid = (HP // RB,)
    return pl.pallas_call(
        _mm2_body,
        grid=grid,
        in_specs=[
            pl.BlockSpec((RB, HID_CH // 2), lambda i: (i, 0)),
            pl.BlockSpec((RB, HID_CH // 2), lambda i: (i, 0)),
            pl.BlockSpec((RB, HID_CH // 2), lambda i: (i, 0)),
            pl.BlockSpec((RB, HID_CH // 2), lambda i: (i, 0)),
            pl.BlockSpec((NW, RB), lambda i: (0, i)),
            pl.BlockSpec((1, HID_CH), lambda i: (0, 0)),
            pl.BlockSpec((HID_CH, OUT_CH), lambda i: (0, 0)),
        ],
        out_specs=pl.BlockSpec((RB, OUT_CH), lambda i: (i, 0)),
        out_shape=jax.ShapeDtypeStruct((HP, OUT_CH), jnp.float32),
    )(a0, a1, h0, h1, deg_parts, b1r, W2)


def _mm3_body(c0_ref, c1_ref, g_ref2, deg_ref, b2_ref, out_ref):
    dinv = _dinv_of(deg_ref[...])
    o = c0_ref[...] + c1_ref[...] + g_ref2[...]
    out_ref[...] = o * dinv[:, None] + b2_ref[...]


def _mm3(c0, c1, g, deg_parts, b2r):
    grid = (HP // RB,)
    return pl.pallas_call(
        _mm3_body,
        grid=grid,
        in_specs=[
            pl.BlockSpec((RB, OUT_CH), lambda i: (i, 0)),
            pl.BlockSpec((RB, OUT_CH), lambda i: (i, 0)),
            pl.BlockSpec((RB, OUT_CH), lambda i: (i, 0)),
            pl.BlockSpec((NW, RB), lambda i: (0, i)),
            pl.BlockSpec((1, OUT_CH), lambda i: (0, 0)),
        ],
        out_specs=pl.BlockSpec((RB, OUT_CH), lambda i: (i, 0)),
        out_shape=jax.ShapeDtypeStruct((HP, OUT_CH), jnp.float32),
    )(c0, c1, g, deg_parts, b2r)


_deg_kernel = _make_deg()
_agg_hid = _make_agg(HID_CH // 2)
_agg_out = _make_agg_edge_split(OUT_CH)


def kernel(x, edge_index, W1, b1, W2, b2):
    src = edge_index[0].astype(jnp.int32)
    dst = edge_index[1].astype(jnp.int32)
    pad_e = EP - N_EDGES
    src_p = jnp.concatenate([src, jnp.zeros((pad_e,), jnp.int32)])
    dst_p = jnp.concatenate([dst, jnp.full((pad_e,), DUMP, jnp.int32)])
    epairs = jnp.stack([src_p.reshape(EP // EC, EC),
                        dst_p.reshape(EP // EC, EC)], axis=1)
    x_p = jnp.pad(x, ((0, HP - N_NODES), (0, 0)))
    zrows = jnp.zeros((HP // NS, OUT_CH), jnp.float32)

    deg_parts = _deg_kernel(dst_p)
    h0, h1 = _mm1(x_p, W1, deg_parts)
    a0, a1 = _agg_hid(h0, h1, zrows, epairs)
    g = _mm2(a0, a1, h0, h1, deg_parts,
             b1.reshape(1, HID_CH), W2)
    c0, c1 = _agg_out(g, zrows, epairs)
    out = _mm3(c0, c1, g, deg_parts,
               b2.reshape(1, OUT_CH))
    return out[:N_NODES]


# speedup vs baseline: 1.0028x; 1.0028x over previous
import functools

import jax
import jax.numpy as jnp
from jax import lax
from jax.experimental import pallas as pl
from jax.experimental.pallas import tpu as pltpu
from jax.experimental.pallas import tpu_sc as plsc

N_NODES = 10000
N_EDGES = 160000
IN_CH = 256
HID_CH = 256
OUT_CH = 128

NC = 2
NS = 16
L = 16
NW = NC * NS

HP = 10240
DUMP = N_NODES
EC = 128
EP = 163840

RB = 512


def _sc_mesh():
    return plsc.VectorSubcoreMesh(core_axis_name="c", subcore_axis_name="s")


def _make_deg():
    epw = EP // NW
    nchunks = epw // EC

    @functools.partial(
        pl.kernel,
        out_type=jax.ShapeDtypeStruct((NW, HP), jnp.float32),
        mesh=_sc_mesh(),
        scratch_types=[
            pltpu.VMEM((EC,), jnp.int32),
            pltpu.VMEM((HP,), jnp.float32),
        ],
        compiler_params=pltpu.CompilerParams(needs_layout_passes=False),
    )
    def deg_kernel(dst_hbm, deg_hbm, dstv, hist):
        c = lax.axis_index("c")
        s = lax.axis_index("s")
        wid = c * NS + s
        zeros = jnp.zeros((L,), jnp.float32)

        def zero_body(i, carry):
            hist[pl.ds(i * L, L)] = zeros
            return carry

        lax.fori_loop(0, HP // L, zero_body, 0)

        ones = jnp.ones((L,), jnp.float32)

        def edge_body(k, carry):
            base = wid * epw + k * EC
            pltpu.sync_copy(dst_hbm.at[pl.ds(base, EC)], dstv)
            for j in range(EC // L):
                idx = dstv[pl.ds(j * L, L)]
                plsc.addupdate_scatter(hist, [idx], ones)
            return carry

        lax.fori_loop(0, nchunks, edge_body, 0)
        pltpu.sync_copy(hist, deg_hbm.at[wid])

    return deg_kernel


def _edge_loop(h, epairs, acc, idxb, rows, sem0, sem1, chunk0, nchunks):
    def load_idx(ch, b):
        pltpu.sync_copy(epairs.at[ch], idxb.at[b])

    def start_gather(b, sem):
        pltpu.async_copy(h.at[idxb.at[b, 0]], rows.at[b], sem)

    def wait_gather(b, sem):
        pltpu.make_async_copy(h.at[idxb.at[b, 0]], rows.at[b], sem).wait()

    def scatter(b):
        pltpu.sync_copy(rows.at[b], acc.at[idxb.at[b, 1]], add=True)

    load_idx(chunk0, 0)
    start_gather(0, sem0)
    nh = nchunks // 2

    def body(m, carry):
        ch = chunk0 + 2 * m
        wait_gather(0, sem0)
        load_idx(ch + 1, 1)
        start_gather(1, sem1)
        scatter(0)
        wait_gather(1, sem1)

        @pl.when(m < nh - 1)
        def _():
            load_idx(ch + 2, 0)
            start_gather(0, sem0)

        scatter(1)
        return carry

    lax.fori_loop(0, nh, body, 0)


def _agg_scratch(F):
    return [
        pltpu.VMEM((2, 2, EC), jnp.int32),
        pltpu.VMEM((2, EC, F), jnp.float32),
        pltpu.VMEM_SHARED((HP, F), jnp.float32),
        pltpu.SemaphoreType.DMA,
        pltpu.SemaphoreType.DMA,
    ]


def _make_agg(F):
    rpw = HP // NS
    nchunks = EP // EC // NS

    @functools.partial(
        pl.kernel,
        out_type=(jax.ShapeDtypeStruct((HP, F), jnp.float32),
                  jax.ShapeDtypeStruct((HP, F), jnp.float32)),
        mesh=_sc_mesh(),
        scratch_types=_agg_scratch(F),
    )
    def agg_kernel(h0, h1, zrows, epairs, o0, o1, idxb, rows, acc,
                   sem0, sem1):
        c = lax.axis_index("c")
        s = lax.axis_index("s")
        r0 = s * rpw
        chunk0 = s * nchunks

        pltpu.sync_copy(zrows, acc.at[pl.ds(r0, rpw)])
        plsc.subcore_barrier()

        @pl.when(c == 0)
        def _():
            _edge_loop(h0, epairs, acc, idxb, rows, sem0, sem1,
                       chunk0, nchunks)

        @pl.when(c == 1)
        def _():
            _edge_loop(h1, epairs, acc, idxb, rows, sem0, sem1,
                       chunk0, nchunks)

        plsc.subcore_barrier()

        def writeback(o):
            pltpu.sync_copy(acc.at[pl.ds(r0, rpw)], o.at[pl.ds(r0, rpw)])

        @pl.when(c == 0)
        def _():
            writeback(o0)

        @pl.when(c == 1)
        def _():
            writeback(o1)

    return agg_kernel


def _make_agg_edge_split(F):
    rpw = HP // NS
    nchunks = EP // EC // NW

    @functools.partial(
        pl.kernel,
        out_type=(jax.ShapeDtypeStruct((HP, F), jnp.float32),
                  jax.ShapeDtypeStruct((HP, F), jnp.float32)),
        mesh=_sc_mesh(),
        scratch_types=_agg_scratch(F),
    )
    def agg_kernel(g, zrows, epairs, o0, o1, idxb, rows, acc, sem0, sem1):
        c = lax.axis_index("c")
        s = lax.axis_index("s")
        wid = c * NS + s
        r0 = s * rpw

        pltpu.sync_copy(zrows, acc.at[pl.ds(r0, rpw)])
        plsc.subcore_barrier()
        _edge_loop(g, epairs, acc, idxb, rows, sem0, sem1,
                   wid * nchunks, nchunks)
        plsc.subcore_barrier()

        @pl.when(c == 0)
        def _():
            pltpu.sync_copy(acc.at[pl.ds(r0, rpw)], o0.at[pl.ds(r0, rpw)])

        @pl.when(c == 1)
        def _():
            pltpu.sync_copy(acc.at[pl.ds(r0, rpw)], o1.at[pl.ds(r0, rpw)])

    return agg_kernel


def _dinv_of(deg_blk):
    return lax.rsqrt(1.0 + jnp.sum(deg_blk, axis=0))


def _mm1_body(x_ref, w_ref, deg_ref, h0_ref, h1_ref):
    dinv = _dinv_of(deg_ref[...])
    h = jnp.dot(x_ref[...], w_ref[...], preferred_element_type=jnp.float32)
    h = h * dinv[:, None]
    h0_ref[...] = h[:, :HID_CH // 2]
    h1_ref[...] = h[:, HID_CH // 2:]


def _mm1(x_p, W1, deg_parts):
    grid = (HP // RB,)
    return pl.pallas_call(
        _mm1_body,
        grid=grid,
        in_specs=[
            pl.BlockSpec((RB, IN_CH), lambda i: (i, 0)),
            pl.BlockSpec((IN_CH, HID_CH), lambda i: (0, 0)),
            pl.BlockSpec((NW, RB), lambda i: (0, i)),
        ],
        out_specs=[
            pl.BlockSpec((RB, HID_CH // 2), lambda i: (i, 0)),
            pl.BlockSpec((RB, HID_CH // 2), lambda i: (i, 0)),
        ],
        out_shape=[
            jax.ShapeDtypeStruct((HP, HID_CH // 2), jnp.float32),
            jax.ShapeDtypeStruct((HP, HID_CH // 2), jnp.float32),
        ],
    )(x_p, W1, deg_parts)


def _mm2_body(a0_ref, a1_ref, h0_ref, h1_ref, deg_ref, b1_ref, w2_ref,
              g_ref):
    dinv = _dinv_of(deg_ref[...])
    b = b1_ref[...]
    t0 = a0_ref[...] + h0_ref[...]
    t1 = a1_ref[...] + h1_ref[...]
    z0 = jnp.maximum(t0 * dinv[:, None] + b[:, :HID_CH // 2], 0.0)
    z1 = jnp.maximum(t1 * dinv[:, None] + b[:, HID_CH // 2:], 0.0)
    w2 = w2_ref[...]
    h = jnp.dot(z0, w2[:HID_CH // 2], preferred_element_type=jnp.float32)
    h = h + jnp.dot(z1, w2[HID_CH // 2:], preferred_element_type=jnp.float32)
    g_ref[...] = h * dinv[:, None]


def _mm2(a0, a1, h0, h1, deg_parts, b1r, W2):
    grid = (HP // RB,)
    return pl.pallas_call(
        _mm2_body,
        grid=grid,
        in_specs=[
            pl.BlockSpec((RB, HID_CH // 2), lambda i: (i, 0)),
            pl.BlockSpec((RB, HID_CH // 2), lambda i: (i, 0)),
            pl.BlockSpec((RB, HID_CH // 2), lambda i: (i, 0)),
            pl.BlockSpec((RB, HID_CH // 2), lambda i: (i, 0)),
            pl.BlockSpec((NW, RB), lambda i: (0, i)),
            pl.BlockSpec((1, HID_CH), lambda i: (0, 0)),
            pl.BlockSpec((HID_CH, OUT_CH), lambda i: (0, 0)),
        ],
        out_specs=pl.BlockSpec((RB, OUT_CH), lambda i: (i, 0)),
        out_shape=jax.ShapeDtypeStruct((HP, OUT_CH), jnp.float32),
    )(a0, a1, h0, h1, deg_parts, b1r, W2)


def _mm3_body(c0_ref, c1_ref, g_ref2, deg_ref, b2_ref, out_ref):
    dinv = _dinv_of(deg_ref[...])
    o = c0_ref[...] + c1_ref[...] + g_ref2[...]
    out_ref[...] = o * dinv[:, None] + b2_ref[...]


def _mm3(c0, c1, g, deg_parts, b2r):
    grid = (HP // RB,)
    return pl.pallas_call(
        _mm3_body,
        grid=grid,
        in_specs=[
            pl.BlockSpec((RB, OUT_CH), lambda i: (i, 0)),
            pl.BlockSpec((RB, OUT_CH), lambda i: (i, 0)),
            pl.BlockSpec((RB, OUT_CH), lambda i: (i, 0)),
            pl.BlockSpec((NW, RB), lambda i: (0, i)),
            pl.BlockSpec((1, OUT_CH), lambda i: (0, 0)),
        ],
        out_specs=pl.BlockSpec((RB, OUT_CH), lambda i: (i, 0)),
        out_shape=jax.ShapeDtypeStruct((HP, OUT_CH), jnp.float32),
    )(c0, c1, g, deg_parts, b2r)


_deg_kernel = _make_deg()
_agg_hid = _make_agg(HID_CH // 2)
_agg_out = _make_agg_edge_split(OUT_CH)


def kernel(x, edge_index, W1, b1, W2, b2):
    src = edge_index[0].astype(jnp.int32)
    dst = edge_index[1].astype(jnp.int32)
    pad_e = EP - N_EDGES
    pad_dst = DUMP + (jnp.arange(pad_e, dtype=jnp.int32) % (HP - N_NODES))
    src_p = jnp.concatenate([src, jnp.zeros((pad_e,), jnp.int32)])
    dst_p = jnp.concatenate([dst, pad_dst])
    epairs = jnp.stack([src_p.reshape(EP // EC, EC),
                        dst_p.reshape(EP // EC, EC)], axis=1)
    x_p = jnp.pad(x, ((0, HP - N_NODES), (0, 0)))
    zrows = jnp.zeros((HP // NS, OUT_CH), jnp.float32)

    deg_parts = _deg_kernel(dst_p)
    h0, h1 = _mm1(x_p, W1, deg_parts)
    a0, a1 = _agg_hid(h0, h1, zrows, epairs)
    g = _mm2(a0, a1, h0, h1, deg_parts,
             b1.reshape(1, HID_CH), W2)
    c0, c1 = _agg_out(g, zrows, epairs)
    out = _mm3(c0, c1, g, deg_parts,
               b2.reshape(1, OUT_CH))
    return out[:N_NODES]


# speedup vs baseline: 2.1855x; 2.1795x over previous
import functools

import jax
import jax.numpy as jnp
from jax import lax
from jax.experimental import pallas as pl
from jax.experimental.pallas import tpu as pltpu
from jax.experimental.pallas import tpu_sc as plsc

N_NODES = 10000
N_EDGES = 160000
IN_CH = 256
HID_CH = 256
OUT_CH = 128

NC = 2
NS = 16
L = 16
NW = NC * NS

HP = 10240
DUMP = N_NODES
EC = 128
EP = 163840

RB = 512


def _sc_mesh():
    return plsc.VectorSubcoreMesh(core_axis_name="c", subcore_axis_name="s")


def _make_deg():
    epw = EP // NW
    nchunks = epw // EC

    @functools.partial(
        pl.kernel,
        out_type=jax.ShapeDtypeStruct((NW, HP), jnp.float32),
        mesh=_sc_mesh(),
        scratch_types=[
            pltpu.VMEM((EC,), jnp.int32),
            pltpu.VMEM((HP,), jnp.float32),
        ],
        compiler_params=pltpu.CompilerParams(needs_layout_passes=False),
    )
    def deg_kernel(dst_hbm, deg_hbm, dstv, hist):
        c = lax.axis_index("c")
        s = lax.axis_index("s")
        wid = c * NS + s
        zeros = jnp.zeros((L,), jnp.float32)

        def zero_body(i, carry):
            hist[pl.ds(i * L, L)] = zeros
            return carry

        lax.fori_loop(0, HP // L, zero_body, 0)

        ones = jnp.ones((L,), jnp.float32)

        def edge_body(k, carry):
            base = wid * epw + k * EC
            pltpu.sync_copy(dst_hbm.at[pl.ds(base, EC)], dstv)
            for j in range(EC // L):
                idx = dstv[pl.ds(j * L, L)]
                plsc.addupdate_scatter(hist, [idx], ones)
            return carry

        lax.fori_loop(0, nchunks, edge_body, 0)
        pltpu.sync_copy(hist, deg_hbm.at[wid])

    return deg_kernel


def _edge_loop(h, epairs, acc, idxb, rows, sem0, sem1, chunk0, nchunks):
    def load_idx(ch, b):
        pltpu.sync_copy(epairs.at[ch], idxb.at[b])

    def start_gather(b, sem):
        pltpu.async_copy(h.at[idxb.at[b, 0]], rows.at[b], sem)

    def wait_gather(b, sem):
        pltpu.make_async_copy(h.at[idxb.at[b, 0]], rows.at[b], sem).wait()

    def scatter(b):
        pltpu.sync_copy(rows.at[b], acc.at[idxb.at[b, 1]], add=True)

    load_idx(chunk0, 0)
    start_gather(0, sem0)
    nh = nchunks // 2

    def body(m, carry):
        ch = chunk0 + 2 * m
        wait_gather(0, sem0)
        load_idx(ch + 1, 1)
        start_gather(1, sem1)
        scatter(0)
        wait_gather(1, sem1)

        @pl.when(m < nh - 1)
        def _():
            load_idx(ch + 2, 0)
            start_gather(0, sem0)

        scatter(1)
        return carry

    lax.fori_loop(0, nh, body, 0)


def _agg_scratch(F):
    return [
        pltpu.VMEM((2, 2, EC), jnp.int32),
        pltpu.VMEM((2, EC, F), jnp.float32),
        pltpu.VMEM_SHARED((HP, F), jnp.float32),
        pltpu.SemaphoreType.DMA,
        pltpu.SemaphoreType.DMA,
    ]


def _make_agg(F):
    rpw = HP // NS
    nchunks = EP // EC // NS

    @functools.partial(
        pl.kernel,
        out_type=(jax.ShapeDtypeStruct((HP, F), jnp.float32),
                  jax.ShapeDtypeStruct((HP, F), jnp.float32)),
        mesh=_sc_mesh(),
        scratch_types=_agg_scratch(F),
    )
    def agg_kernel(h0, h1, zrows, epairs, o0, o1, idxb, rows, acc,
                   sem0, sem1):
        c = lax.axis_index("c")
        s = lax.axis_index("s")
        r0 = s * rpw
        chunk0 = s * nchunks

        pltpu.sync_copy(zrows, acc.at[pl.ds(r0, rpw)])
        plsc.subcore_barrier()

        @pl.when(c == 0)
        def _():
            _edge_loop(h0, epairs, acc, idxb, rows, sem0, sem1,
                       chunk0, nchunks)

        @pl.when(c == 1)
        def _():
            _edge_loop(h1, epairs, acc, idxb, rows, sem0, sem1,
                       chunk0, nchunks)

        plsc.subcore_barrier()

        def writeback(o):
            pltpu.sync_copy(acc.at[pl.ds(r0, rpw)], o.at[pl.ds(r0, rpw)])

        @pl.when(c == 0)
        def _():
            writeback(o0)

        @pl.when(c == 1)
        def _():
            writeback(o1)

    return agg_kernel


def _make_agg_edge_split(F):
    rpw = HP // NS
    nchunks = EP // EC // NW

    @functools.partial(
        pl.kernel,
        out_type=(jax.ShapeDtypeStruct((HP, F), jnp.float32),
                  jax.ShapeDtypeStruct((HP, F), jnp.float32)),
        mesh=_sc_mesh(),
        scratch_types=_agg_scratch(F),
    )
    def agg_kernel(g, zrows, epairs, o0, o1, idxb, rows, acc, sem0, sem1):
        c = lax.axis_index("c")
        s = lax.axis_index("s")
        wid = c * NS + s
        r0 = s * rpw

        pltpu.sync_copy(zrows, acc.at[pl.ds(r0, rpw)])
        plsc.subcore_barrier()
        _edge_loop(g, epairs, acc, idxb, rows, sem0, sem1,
                   wid * nchunks, nchunks)
        plsc.subcore_barrier()

        @pl.when(c == 0)
        def _():
            pltpu.sync_copy(acc.at[pl.ds(r0, rpw)], o0.at[pl.ds(r0, rpw)])

        @pl.when(c == 1)
        def _():
            pltpu.sync_copy(acc.at[pl.ds(r0, rpw)], o1.at[pl.ds(r0, rpw)])

    return agg_kernel


def _dinv_of(deg_blk):
    return lax.rsqrt(1.0 + jnp.sum(deg_blk, axis=0))


def _mm1_body(x_ref, w_ref, deg_ref, h0_ref, h1_ref):
    dinv = _dinv_of(deg_ref[...])
    h = jnp.dot(x_ref[...], w_ref[...], preferred_element_type=jnp.float32)
    h = h * dinv[:, None]
    h0_ref[...] = h[:, :HID_CH // 2]
    h1_ref[...] = h[:, HID_CH // 2:]


def _mm1(x_p, W1, deg_parts):
    grid = (HP // RB,)
    return pl.pallas_call(
        _mm1_body,
        grid=grid,
        in_specs=[
            pl.BlockSpec((RB, IN_CH), lambda i: (i, 0)),
            pl.BlockSpec((IN_CH, HID_CH), lambda i: (0, 0)),
            pl.BlockSpec((NW, RB), lambda i: (0, i)),
        ],
        out_specs=[
            pl.BlockSpec((RB, HID_CH // 2), lambda i: (i, 0)),
            pl.BlockSpec((RB, HID_CH // 2), lambda i: (i, 0)),
        ],
        out_shape=[
            jax.ShapeDtypeStruct((HP, HID_CH // 2), jnp.float32),
            jax.ShapeDtypeStruct((HP, HID_CH // 2), jnp.float32),
        ],
    )(x_p, W1, deg_parts)


def _mm2_body(a0_ref, a1_ref, h0_ref, h1_ref, deg_ref, b1_ref, w2_ref,
              g_ref):
    dinv = _dinv_of(deg_ref[...])
    b = b1_ref[...]
    t0 = a0_ref[...] + h0_ref[...]
    t1 = a1_ref[...] + h1_ref[...]
    z0 = jnp.maximum(t0 * dinv[:, None] + b[:, :HID_CH // 2], 0.0)
    z1 = jnp.maximum(t1 * dinv[:, None] + b[:, HID_CH // 2:], 0.0)
    w2 = w2_ref[...]
    h = jnp.dot(z0, w2[:HID_CH // 2], preferred_element_type=jnp.float32)
    h = h + jnp.dot(z1, w2[HID_CH // 2:], preferred_element_type=jnp.float32)
    g_ref[...] = h * dinv[:, None]


def _mm2(a0, a1, h0, h1, deg_parts, b1r, W2):
    grid = (HP // RB,)
    return pl.pallas_call(
        _mm2_body,
        grid=grid,
        in_specs=[
            pl.BlockSpec((RB, HID_CH // 2), lambda i: (i, 0)),
            pl.BlockSpec((RB, HID_CH // 2), lambda i: (i, 0)),
            pl.BlockSpec((RB, HID_CH // 2), lambda i: (i, 0)),
            pl.BlockSpec((RB, HID_CH // 2), lambda i: (i, 0)),
            pl.BlockSpec((NW, RB), lambda i: (0, i)),
            pl.BlockSpec((1, HID_CH), lambda i: (0, 0)),
            pl.BlockSpec((HID_CH, OUT_CH), lambda i: (0, 0)),
        ],
        out_specs=pl.BlockSpec((RB, OUT_CH), lambda i: (i, 0)),
        out_shape=jax.ShapeDtypeStruct((HP, OUT_CH), jnp.float32),
    )(a0, a1, h0, h1, deg_parts, b1r, W2)


def _mm3_body(c0_ref, c1_ref, g_ref2, deg_ref, b2_ref, out_ref):
    dinv = _dinv_of(deg_ref[...])
    o = c0_ref[...] + c1_ref[...] + g_ref2[...]
    out_ref[...] = o * dinv[:, None] + b2_ref[...]


def _mm3(c0, c1, g, deg_parts, b2r):
    grid = (HP // RB,)
    return pl.pallas_call(
        _mm3_body,
        grid=grid,
        in_specs=[
            pl.BlockSpec((RB, OUT_CH), lambda i: (i, 0)),
            pl.BlockSpec((RB, OUT_CH), lambda i: (i, 0)),
            pl.BlockSpec((RB, OUT_CH), lambda i: (i, 0)),
            pl.BlockSpec((NW, RB), lambda i: (0, i)),
            pl.BlockSpec((1, OUT_CH), lambda i: (0, 0)),
        ],
        out_specs=pl.BlockSpec((RB, OUT_CH), lambda i: (i, 0)),
        out_shape=jax.ShapeDtypeStruct((HP, OUT_CH), jnp.float32),
    )(c0, c1, g, deg_parts, b2r)


_deg_kernel = _make_deg()
_agg_hid = _make_agg(HID_CH // 2)
_agg_out = _make_agg_edge_split(OUT_CH)


def kernel(x, edge_index, W1, b1, W2, b2):
    src = edge_index[0].astype(jnp.int32)
    dst = edge_index[1].astype(jnp.int32)
    pad_e = EP - N_EDGES
    spare = HP - N_NODES
    pad_iota = jnp.arange(pad_e, dtype=jnp.int32)
    pad_dst = DUMP + (pad_iota % spare)
    pad_src = DUMP + ((pad_iota + spare // 2) % spare)
    src_p = jnp.concatenate([src, pad_src])
    dst_p = jnp.concatenate([dst, pad_dst])
    epairs = jnp.stack([src_p.reshape(EP // EC, EC),
                        dst_p.reshape(EP // EC, EC)], axis=1)
    x_p = jnp.pad(x, ((0, HP - N_NODES), (0, 0)))
    zrows = jnp.zeros((HP // NS, OUT_CH), jnp.float32)

    deg_parts = _deg_kernel(dst_p)
    h0, h1 = _mm1(x_p, W1, deg_parts)
    a0, a1 = _agg_hid(h0, h1, zrows, epairs)
    g = _mm2(a0, a1, h0, h1, deg_parts,
             b1.reshape(1, HID_CH), W2)
    c0, c1 = _agg_out(g, zrows, epairs)
    out = _mm3(c0, c1, g, deg_parts,
               b2.reshape(1, OUT_CH))
    return out[:N_NODES]


# speedup vs baseline: 2.3176x; 1.0605x over previous
import functools

import jax
import jax.numpy as jnp
from jax import lax
from jax.experimental import pallas as pl
from jax.experimental.pallas import tpu as pltpu
from jax.experimental.pallas import tpu_sc as plsc

N_NODES = 10000
N_EDGES = 160000
IN_CH = 256
HID_CH = 256
OUT_CH = 128

NC = 2
NS = 16
L = 16
NW = NC * NS

HP = 10240
DUMP = N_NODES
EC = 128
EP = 163840

RB = 1024


def _sc_mesh():
    return plsc.VectorSubcoreMesh(core_axis_name="c", subcore_axis_name="s")


def _make_deg():
    epw = EP // NW
    nchunks = epw // EC

    @functools.partial(
        pl.kernel,
        out_type=jax.ShapeDtypeStruct((NW, HP), jnp.float32),
        mesh=_sc_mesh(),
        scratch_types=[
            pltpu.VMEM((EC,), jnp.int32),
            pltpu.VMEM((HP,), jnp.float32),
        ],
        compiler_params=pltpu.CompilerParams(needs_layout_passes=False),
    )
    def deg_kernel(dst_hbm, deg_hbm, dstv, hist):
        c = lax.axis_index("c")
        s = lax.axis_index("s")
        wid = c * NS + s
        zeros = jnp.zeros((L,), jnp.float32)

        def zero_body(i, carry):
            hist[pl.ds(i * L, L)] = zeros
            return carry

        lax.fori_loop(0, HP // L, zero_body, 0)

        ones = jnp.ones((L,), jnp.float32)

        def edge_body(k, carry):
            base = wid * epw + k * EC
            pltpu.sync_copy(dst_hbm.at[pl.ds(base, EC)], dstv)
            for j in range(EC // L):
                idx = dstv[pl.ds(j * L, L)]
                plsc.addupdate_scatter(hist, [idx], ones)
            return carry

        lax.fori_loop(0, nchunks, edge_body, 0)
        pltpu.sync_copy(hist, deg_hbm.at[wid])

    return deg_kernel


def _edge_loop(h, epairs, acc, idxb, rows, sem0, sem1, chunk0, nchunks):
    def load_idx(ch, b):
        pltpu.sync_copy(epairs.at[ch], idxb.at[b])

    def start_gather(b, sem):
        pltpu.async_copy(h.at[idxb.at[b, 0]], rows.at[b], sem)

    def wait_gather(b, sem):
        pltpu.make_async_copy(h.at[idxb.at[b, 0]], rows.at[b], sem).wait()

    def scatter(b):
        pltpu.sync_copy(rows.at[b], acc.at[idxb.at[b, 1]], add=True)

    load_idx(chunk0, 0)
    start_gather(0, sem0)
    nh = nchunks // 2

    def body(m, carry):
        ch = chunk0 + 2 * m
        wait_gather(0, sem0)
        load_idx(ch + 1, 1)
        start_gather(1, sem1)
        scatter(0)
        wait_gather(1, sem1)

        @pl.when(m < nh - 1)
        def _():
            load_idx(ch + 2, 0)
            start_gather(0, sem0)

        scatter(1)
        return carry

    lax.fori_loop(0, nh, body, 0)


def _agg_scratch(F):
    return [
        pltpu.VMEM((2, 2, EC), jnp.int32),
        pltpu.VMEM((2, EC, F), jnp.float32),
        pltpu.VMEM_SHARED((HP, F), jnp.float32),
        pltpu.SemaphoreType.DMA,
        pltpu.SemaphoreType.DMA,
    ]


def _make_agg(F):
    rpw = HP // NS
    nchunks = EP // EC // NS

    @functools.partial(
        pl.kernel,
        out_type=(jax.ShapeDtypeStruct((HP, F), jnp.float32),
                  jax.ShapeDtypeStruct((HP, F), jnp.float32)),
        mesh=_sc_mesh(),
        scratch_types=_agg_scratch(F),
    )
    def agg_kernel(h0, h1, zrows, epairs, o0, o1, idxb, rows, acc,
                   sem0, sem1):
        c = lax.axis_index("c")
        s = lax.axis_index("s")
        r0 = s * rpw
        chunk0 = s * nchunks

        pltpu.sync_copy(zrows, acc.at[pl.ds(r0, rpw)])
        plsc.subcore_barrier()

        @pl.when(c == 0)
        def _():
            _edge_loop(h0, epairs, acc, idxb, rows, sem0, sem1,
                       chunk0, nchunks)

        @pl.when(c == 1)
        def _():
            _edge_loop(h1, epairs, acc, idxb, rows, sem0, sem1,
                       chunk0, nchunks)

        plsc.subcore_barrier()

        def writeback(o):
            pltpu.sync_copy(acc.at[pl.ds(r0, rpw)], o.at[pl.ds(r0, rpw)])

        @pl.when(c == 0)
        def _():
            writeback(o0)

        @pl.when(c == 1)
        def _():
            writeback(o1)

    return agg_kernel


def _make_agg_edge_split(F):
    rpw = HP // NS
    nchunks = EP // EC // NW

    @functools.partial(
        pl.kernel,
        out_type=(jax.ShapeDtypeStruct((HP, F), jnp.float32),
                  jax.ShapeDtypeStruct((HP, F), jnp.float32)),
        mesh=_sc_mesh(),
        scratch_types=_agg_scratch(F),
    )
    def agg_kernel(g, zrows, epairs, o0, o1, idxb, rows, acc, sem0, sem1):
        c = lax.axis_index("c")
        s = lax.axis_index("s")
        wid = c * NS + s
        r0 = s * rpw

        pltpu.sync_copy(zrows, acc.at[pl.ds(r0, rpw)])
        plsc.subcore_barrier()
        _edge_loop(g, epairs, acc, idxb, rows, sem0, sem1,
                   wid * nchunks, nchunks)
        plsc.subcore_barrier()

        @pl.when(c == 0)
        def _():
            pltpu.sync_copy(acc.at[pl.ds(r0, rpw)], o0.at[pl.ds(r0, rpw)])

        @pl.when(c == 1)
        def _():
            pltpu.sync_copy(acc.at[pl.ds(r0, rpw)], o1.at[pl.ds(r0, rpw)])

    return agg_kernel


def _dinv_of(deg_blk):
    return lax.rsqrt(1.0 + jnp.sum(deg_blk, axis=0))


def _mm1_body(x_ref, w_ref, deg_ref, h0_ref, h1_ref):
    dinv = _dinv_of(deg_ref[...])
    h = jnp.dot(x_ref[...], w_ref[...], preferred_element_type=jnp.float32)
    h = h * dinv[:, None]
    h0_ref[...] = h[:, :HID_CH // 2]
    h1_ref[...] = h[:, HID_CH // 2:]


def _mm1(x_p, W1, deg_parts):
    grid = (HP // RB,)
    return pl.pallas_call(
        _mm1_body,
        grid=grid,
        in_specs=[
            pl.BlockSpec((RB, IN_CH), lambda i: (i, 0)),
            pl.BlockSpec((IN_CH, HID_CH), lambda i: (0, 0)),
            pl.BlockSpec((NW, RB), lambda i: (0, i)),
        ],
        out_specs=[
            pl.BlockSpec((RB, HID_CH // 2), lambda i: (i, 0)),
            pl.BlockSpec((RB, HID_CH // 2), lambda i: (i, 0)),
        ],
        out_shape=[
            jax.ShapeDtypeStruct((HP, HID_CH // 2), jnp.float32),
            jax.ShapeDtypeStruct((HP, HID_CH // 2), jnp.float32),
        ],
    )(x_p, W1, deg_parts)


def _mm2_body(a0_ref, a1_ref, h0_ref, h1_ref, deg_ref, b1_ref, w2_ref,
              g_ref):
    dinv = _dinv_of(deg_ref[...])
    b = b1_ref[...]
    t0 = a0_ref[...] + h0_ref[...]
    t1 = a1_ref[...] + h1_ref[...]
    z0 = jnp.maximum(t0 * dinv[:, None] + b[:, :HID_CH // 2], 0.0)
    z1 = jnp.maximum(t1 * dinv[:, None] + b[:, HID_CH // 2:], 0.0)
    w2 = w2_ref[...]
    h = jnp.dot(z0, w2[:HID_CH // 2], preferred_element_type=jnp.float32)
    h = h + jnp.dot(z1, w2[HID_CH // 2:], preferred_element_type=jnp.float32)
    g_ref[...] = h * dinv[:, None]


def _mm2(a0, a1, h0, h1, deg_parts, b1r, W2):
    grid = (HP // RB,)
    return pl.pallas_call(
        _mm2_body,
        grid=grid,
        in_specs=[
            pl.BlockSpec((RB, HID_CH // 2), lambda i: (i, 0)),
            pl.BlockSpec((RB, HID_CH // 2), lambda i: (i, 0)),
            pl.BlockSpec((RB, HID_CH // 2), lambda i: (i, 0)),
            pl.BlockSpec((RB, HID_CH // 2), lambda i: (i, 0)),
            pl.BlockSpec((NW, RB), lambda i: (0, i)),
            pl.BlockSpec((1, HID_CH), lambda i: (0, 0)),
            pl.BlockSpec((HID_CH, OUT_CH), lambda i: (0, 0)),
        ],
        out_specs=pl.BlockSpec((RB, OUT_CH), lambda i: (i, 0)),
        out_shape=jax.ShapeDtypeStruct((HP, OUT_CH), jnp.float32),
    )(a0, a1, h0, h1, deg_parts, b1r, W2)


def _mm3_body(c0_ref, c1_ref, g_ref2, deg_ref, b2_ref, out_ref):
    dinv = _dinv_of(deg_ref[...])
    o = c0_ref[...] + c1_ref[...] + g_ref2[...]
    out_ref[...] = o * dinv[:, None] + b2_ref[...]


def _mm3(c0, c1, g, deg_parts, b2r):
    grid = (HP // RB,)
    return pl.pallas_call(
        _mm3_body,
        grid=grid,
        in_specs=[
            pl.BlockSpec((RB, OUT_CH), lambda i: (i, 0)),
            pl.BlockSpec((RB, OUT_CH), lambda i: (i, 0)),
            pl.BlockSpec((RB, OUT_CH), lambda i: (i, 0)),
            pl.BlockSpec((NW, RB), lambda i: (0, i)),
            pl.BlockSpec((1, OUT_CH), lambda i: (0, 0)),
        ],
        out_specs=pl.BlockSpec((RB, OUT_CH), lambda i: (i, 0)),
        out_shape=jax.ShapeDtypeStruct((N_NODES, OUT_CH), jnp.float32),
    )(c0, c1, g, deg_parts, b2r)


_deg_kernel = _make_deg()
_agg_hid = _make_agg(HID_CH // 2)
_agg_out = _make_agg_edge_split(OUT_CH)


def kernel(x, edge_index, W1, b1, W2, b2):
    src = edge_index[0].astype(jnp.int32)
    dst = edge_index[1].astype(jnp.int32)
    pad_e = EP - N_EDGES
    spare = HP - N_NODES
    pad_iota = jnp.arange(pad_e, dtype=jnp.int32)
    pad_dst = DUMP + (pad_iota % spare)
    pad_src = DUMP + ((pad_iota + spare // 2) % spare)
    src_p = jnp.concatenate([src, pad_src])
    dst_p = jnp.concatenate([dst, pad_dst])
    epairs = jnp.stack([src_p.reshape(EP // EC, EC),
                        dst_p.reshape(EP // EC, EC)], axis=1)
    zrows = jnp.zeros((HP // NS, OUT_CH), jnp.float32)

    deg_parts = _deg_kernel(dst_p)
    h0, h1 = _mm1(x, W1, deg_parts)
    a0, a1 = _agg_hid(h0, h1, zrows, epairs)
    g = _mm2(a0, a1, h0, h1, deg_parts,
             b1.reshape(1, HID_CH), W2)
    c0, c1 = _agg_out(g, zrows, epairs)
    return _mm3(c0, c1, g, deg_parts,
                b2.reshape(1, OUT_CH))


# speedup vs baseline: 2.9483x; 1.2721x over previous
import functools

import jax
import jax.numpy as jnp
from jax import lax
from jax.experimental import pallas as pl
from jax.experimental.pallas import tpu as pltpu
from jax.experimental.pallas import tpu_sc as plsc

N_NODES = 10000
N_EDGES = 160000
IN_CH = 256
HID_CH = 256
OUT_CH = 128

NC = 2
NS = 16
L = 16
NW = NC * NS

HP = 10240
DUMP = N_NODES
EC = 128
EP = 163840

RB = 1024


def _sc_mesh():
    return plsc.VectorSubcoreMesh(core_axis_name="c", subcore_axis_name="s")


def _make_deg():
    epw = EP // NW
    nchunks = epw // EC

    @functools.partial(
        pl.kernel,
        out_type=jax.ShapeDtypeStruct((NW, HP), jnp.float32),
        mesh=_sc_mesh(),
        scratch_types=[
            pltpu.VMEM((2, G * EC), jnp.int32),
            pltpu.VMEM((HP,), jnp.float32),
            pltpu.SemaphoreType.DMA,
        ],
        compiler_params=pltpu.CompilerParams(needs_layout_passes=False),
    )
    def deg_kernel(dst_hbm, deg_hbm, dstb, hist, isem):
        c = lax.axis_index("c")
        s = lax.axis_index("s")
        wid = c * NS + s
        base = wid * epw
        ng = nchunks // G
        zeros = jnp.zeros((L,), jnp.float32)

        def zero_body(i, carry):
            for j in range(8):
                hist[pl.ds((i * 8 + j) * L, L)] = zeros
            return carry

        lax.fori_loop(0, HP // L // 8, zero_body, 0)

        ones = jnp.ones((L,), jnp.float32)

        def idx_start(g, gb):
            pltpu.async_copy(dst_hbm.at[pl.ds(base + g * G * EC, G * EC)],
                             dstb.at[gb], isem)

        def idx_wait(gb):
            pltpu.make_async_copy(dst_hbm.at[pl.ds(base, G * EC)],
                                  dstb.at[gb], isem).wait()

        pltpu.sync_copy(dst_hbm.at[pl.ds(base, G * EC)], dstb.at[0])
        idx_start(1, 1)

        def group(m, gb):
            for j in range(G * EC // L):
                idx = dstb[gb, pl.ds(j * L, L)]
                plsc.addupdate_scatter(hist, [idx], ones)

            @pl.when(m < ng - 1)
            def _():
                idx_wait(gb ^ 1)

            @pl.when(m < ng - 2)
            def _():
                idx_start(m + 2, gb)

        def body(mm, carry):
            group(2 * mm, 0)
            group(2 * mm + 1, 1)
            return carry

        lax.fori_loop(0, ng // 2, body, 0)
        pltpu.sync_copy(hist, deg_hbm.at[wid])

    return deg_kernel


G = 10


def _edge_loop(h, epairs, acc, idxb, rows, gsems, isem, chunk0, nchunks):
    ng = nchunks // G

    def idx_start(g, gb):
        pltpu.async_copy(epairs.at[pl.ds(chunk0 + g * G, G)], idxb.at[gb],
                         isem)

    def idx_wait(gb):
        pltpu.make_async_copy(epairs.at[pl.ds(chunk0, G)], idxb.at[gb],
                              isem).wait()

    def start_gather(b, gb, j):
        pltpu.async_copy(h.at[idxb.at[gb, j, 0]], rows.at[b], gsems[b])

    def wait_gather(b):
        pltpu.make_async_copy(h.at[idxb.at[0, 0, 0]], rows.at[b],
                              gsems[b]).wait()

    def scatter(b, gb, j):
        pltpu.sync_copy(rows.at[b], acc.at[idxb.at[gb, j, 1]], add=True)

    pltpu.sync_copy(epairs.at[pl.ds(chunk0, G)], idxb.at[0])
    idx_start(1, 1)
    start_gather(0, 0, 0)

    def group(m, gb):
        for j in range(G):
            rb = j & 1
            wait_gather(rb)
            if j < G - 1:
                start_gather(rb ^ 1, gb, j + 1)
            else:
                @pl.when(m < ng - 1)
                def _():
                    idx_wait(gb ^ 1)
                    start_gather(rb ^ 1, gb ^ 1, 0)
            scatter(rb, gb, j)

        @pl.when(m < ng - 2)
        def _():
            idx_start(m + 2, gb)

    def body(mm, carry):
        group(2 * mm, 0)
        group(2 * mm + 1, 1)
        return carry

    lax.fori_loop(0, ng // 2, body, 0)


def _agg_scratch(F):
    return [
        pltpu.VMEM((2, G, 2, EC), jnp.int32),
        pltpu.VMEM((2, EC, F), jnp.float32),
        pltpu.VMEM_SHARED((HP, F), jnp.float32),
        (pltpu.SemaphoreType.DMA, pltpu.SemaphoreType.DMA),
        pltpu.SemaphoreType.DMA,
    ]


def _make_agg(F):
    rpw = HP // NS
    nchunks = EP // EC // NS

    @functools.partial(
        pl.kernel,
        out_type=(jax.ShapeDtypeStruct((HP, F), jnp.float32),
                  jax.ShapeDtypeStruct((HP, F), jnp.float32)),
        mesh=_sc_mesh(),
        scratch_types=_agg_scratch(F),
    )
    def agg_kernel(h0, h1, zrows, epairs, o0, o1, idxb, rows, acc,
                   gsems, isem):
        c = lax.axis_index("c")
        s = lax.axis_index("s")
        r0 = s * rpw
        chunk0 = s * nchunks

        pltpu.sync_copy(zrows, acc.at[pl.ds(r0, rpw)])
        plsc.subcore_barrier()

        @pl.when(c == 0)
        def _():
            _edge_loop(h0, epairs, acc, idxb, rows, gsems, isem,
                       chunk0, nchunks)

        @pl.when(c == 1)
        def _():
            _edge_loop(h1, epairs, acc, idxb, rows, gsems, isem,
                       chunk0, nchunks)

        plsc.subcore_barrier()

        def writeback(o):
            pltpu.sync_copy(acc.at[pl.ds(r0, rpw)], o.at[pl.ds(r0, rpw)])

        @pl.when(c == 0)
        def _():
            writeback(o0)

        @pl.when(c == 1)
        def _():
            writeback(o1)

    return agg_kernel


def _make_agg_edge_split(F):
    rpw = HP // NS
    nchunks = EP // EC // NW

    @functools.partial(
        pl.kernel,
        out_type=(jax.ShapeDtypeStruct((HP, F), jnp.float32),
                  jax.ShapeDtypeStruct((HP, F), jnp.float32)),
        mesh=_sc_mesh(),
        scratch_types=_agg_scratch(F),
    )
    def agg_kernel(g, zrows, epairs, o0, o1, idxb, rows, acc, gsems, isem):
        c = lax.axis_index("c")
        s = lax.axis_index("s")
        wid = c * NS + s
        r0 = s * rpw

        pltpu.sync_copy(zrows, acc.at[pl.ds(r0, rpw)])
        plsc.subcore_barrier()
        _edge_loop(g, epairs, acc, idxb, rows, gsems, isem,
                   wid * nchunks, nchunks)
        plsc.subcore_barrier()

        @pl.when(c == 0)
        def _():
            pltpu.sync_copy(acc.at[pl.ds(r0, rpw)], o0.at[pl.ds(r0, rpw)])

        @pl.when(c == 1)
        def _():
            pltpu.sync_copy(acc.at[pl.ds(r0, rpw)], o1.at[pl.ds(r0, rpw)])

    return agg_kernel


def _dinv_of(deg_blk):
    return lax.rsqrt(1.0 + jnp.sum(deg_blk, axis=0))


def _mm1_body(x_ref, w_ref, deg_ref, h0_ref, h1_ref):
    dinv = _dinv_of(deg_ref[...])
    h = jnp.dot(x_ref[...], w_ref[...], preferred_element_type=jnp.float32)
    h = h * dinv[:, None]
    h0_ref[...] = h[:, :HID_CH // 2]
    h1_ref[...] = h[:, HID_CH // 2:]


def _mm1(x_p, W1, deg_parts):
    grid = (HP // RB,)
    return pl.pallas_call(
        _mm1_body,
        grid=grid,
        in_specs=[
            pl.BlockSpec((RB, IN_CH), lambda i: (i, 0)),
            pl.BlockSpec((IN_CH, HID_CH), lambda i: (0, 0)),
            pl.BlockSpec((NW, RB), lambda i: (0, i)),
        ],
        out_specs=[
            pl.BlockSpec((RB, HID_CH // 2), lambda i: (i, 0)),
            pl.BlockSpec((RB, HID_CH // 2), lambda i: (i, 0)),
        ],
        out_shape=[
            jax.ShapeDtypeStruct((HP, HID_CH // 2), jnp.float32),
            jax.ShapeDtypeStruct((HP, HID_CH // 2), jnp.float32),
        ],
    )(x_p, W1, deg_parts)


def _mm2_body(a0_ref, a1_ref, h0_ref, h1_ref, deg_ref, b1_ref, w2_ref,
              g_ref):
    dinv = _dinv_of(deg_ref[...])
    b = b1_ref[...]
    t0 = a0_ref[...] + h0_ref[...]
    t1 = a1_ref[...] + h1_ref[...]
    z0 = jnp.maximum(t0 * dinv[:, None] + b[:, :HID_CH // 2], 0.0)
    z1 = jnp.maximum(t1 * dinv[:, None] + b[:, HID_CH // 2:], 0.0)
    w2 = w2_ref[...]
    h = jnp.dot(z0, w2[:HID_CH // 2], preferred_element_type=jnp.float32)
    h = h + jnp.dot(z1, w2[HID_CH // 2:], preferred_element_type=jnp.float32)
    g_ref[...] = h * dinv[:, None]


def _mm2(a0, a1, h0, h1, deg_parts, b1r, W2):
    grid = (HP // RB,)
    return pl.pallas_call(
        _mm2_body,
        grid=grid,
        in_specs=[
            pl.BlockSpec((RB, HID_CH // 2), lambda i: (i, 0)),
            pl.BlockSpec((RB, HID_CH // 2), lambda i: (i, 0)),
            pl.BlockSpec((RB, HID_CH // 2), lambda i: (i, 0)),
            pl.BlockSpec((RB, HID_CH // 2), lambda i: (i, 0)),
            pl.BlockSpec((NW, RB), lambda i: (0, i)),
            pl.BlockSpec((1, HID_CH), lambda i: (0, 0)),
            pl.BlockSpec((HID_CH, OUT_CH), lambda i: (0, 0)),
        ],
        out_specs=pl.BlockSpec((RB, OUT_CH), lambda i: (i, 0)),
        out_shape=jax.ShapeDtypeStruct((HP, OUT_CH), jnp.float32),
    )(a0, a1, h0, h1, deg_parts, b1r, W2)


def _mm3_body(c0_ref, c1_ref, g_ref2, deg_ref, b2_ref, out_ref):
    dinv = _dinv_of(deg_ref[...])
    o = c0_ref[...] + c1_ref[...] + g_ref2[...]
    out_ref[...] = o * dinv[:, None] + b2_ref[...]


def _mm3(c0, c1, g, deg_parts, b2r):
    grid = (HP // RB,)
    return pl.pallas_call(
        _mm3_body,
        grid=grid,
        in_specs=[
            pl.BlockSpec((RB, OUT_CH), lambda i: (i, 0)),
            pl.BlockSpec((RB, OUT_CH), lambda i: (i, 0)),
            pl.BlockSpec((RB, OUT_CH), lambda i: (i, 0)),
            pl.BlockSpec((NW, RB), lambda i: (0, i)),
            pl.BlockSpec((1, OUT_CH), lambda i: (0, 0)),
        ],
        out_specs=pl.BlockSpec((RB, OUT_CH), lambda i: (i, 0)),
        out_shape=jax.ShapeDtypeStruct((N_NODES, OUT_CH), jnp.float32),
    )(c0, c1, g, deg_parts, b2r)


_deg_kernel = _make_deg()
_agg_hid = _make_agg(HID_CH // 2)
_agg_out = _make_agg_edge_split(OUT_CH)


def kernel(x, edge_index, W1, b1, W2, b2):
    src = edge_index[0].astype(jnp.int32)
    dst = edge_index[1].astype(jnp.int32)
    pad_e = EP - N_EDGES
    spare = HP - N_NODES
    pad_iota = jnp.arange(pad_e, dtype=jnp.int32)
    pad_dst = DUMP + (pad_iota % spare)
    pad_src = DUMP + ((pad_iota + spare // 2) % spare)
    src_p = jnp.concatenate([src, pad_src])
    dst_p = jnp.concatenate([dst, pad_dst])
    epairs = jnp.stack([src_p.reshape(EP // EC, EC),
                        dst_p.reshape(EP // EC, EC)], axis=1)
    zrows = jnp.zeros((HP // NS, OUT_CH), jnp.float32)

    deg_parts = _deg_kernel(dst_p)
    h0, h1 = _mm1(x, W1, deg_parts)
    a0, a1 = _agg_hid(h0, h1, zrows, epairs)
    g = _mm2(a0, a1, h0, h1, deg_parts,
             b1.reshape(1, HID_CH), W2)
    c0, c1 = _agg_out(g, zrows, epairs)
    return _mm3(c0, c1, g, deg_parts,
                b2.reshape(1, OUT_CH))


# speedup vs baseline: 2.9495x; 1.0004x over previous
import functools

import jax
import jax.numpy as jnp
from jax import lax
from jax.experimental import pallas as pl
from jax.experimental.pallas import tpu as pltpu
from jax.experimental.pallas import tpu_sc as plsc

N_NODES = 10000
N_EDGES = 160000
IN_CH = 256
HID_CH = 256
OUT_CH = 128

NC = 2
NS = 16
L = 16
NW = NC * NS

HP = 10240
DUMP = N_NODES
EC = 128
EP = 163840

RB = 1024


def _sc_mesh():
    return plsc.VectorSubcoreMesh(core_axis_name="c", subcore_axis_name="s")


def _make_deg():
    epw = EP // NW
    nchunks = epw // EC

    @functools.partial(
        pl.kernel,
        out_type=jax.ShapeDtypeStruct((NW, HP), jnp.float32),
        mesh=_sc_mesh(),
        scratch_types=[
            pltpu.VMEM((2, G * EC), jnp.int32),
            pltpu.VMEM((HP,), jnp.float32),
            pltpu.SemaphoreType.DMA,
        ],
        compiler_params=pltpu.CompilerParams(needs_layout_passes=False),
    )
    def deg_kernel(dst_hbm, deg_hbm, dstb, hist, isem):
        c = lax.axis_index("c")
        s = lax.axis_index("s")
        wid = c * NS + s
        base = wid * epw
        ng = nchunks // G
        zeros = jnp.zeros((L,), jnp.float32)

        def zero_body(i, carry):
            for j in range(8):
                hist[pl.ds((i * 8 + j) * L, L)] = zeros
            return carry

        lax.fori_loop(0, HP // L // 8, zero_body, 0)

        ones = jnp.ones((L,), jnp.float32)

        def idx_start(g, gb):
            pltpu.async_copy(dst_hbm.at[pl.ds(base + g * G * EC, G * EC)],
                             dstb.at[gb], isem)

        def idx_wait(gb):
            pltpu.make_async_copy(dst_hbm.at[pl.ds(base, G * EC)],
                                  dstb.at[gb], isem).wait()

        pltpu.sync_copy(dst_hbm.at[pl.ds(base, G * EC)], dstb.at[0])
        idx_start(1, 1)

        def group(m, gb):
            for j in range(G * EC // L):
                idx = dstb[gb, pl.ds(j * L, L)]
                plsc.addupdate_scatter(hist, [idx], ones)

            @pl.when(m < ng - 1)
            def _():
                idx_wait(gb ^ 1)

            @pl.when(m < ng - 2)
            def _():
                idx_start(m + 2, gb)

        def body(mm, carry):
            group(2 * mm, 0)
            group(2 * mm + 1, 1)
            return carry

        lax.fori_loop(0, ng // 2, body, 0)
        pltpu.sync_copy(hist, deg_hbm.at[wid])

    return deg_kernel


G = 10


def _edge_loop(h, epairs, acc, idxb, rows, gsem0, gsem1, isem,
               chunk0, nchunks):
    gsems = (gsem0, gsem1)
    ng = nchunks // G

    def idx_start(g, gb):
        pltpu.async_copy(epairs.at[pl.ds(chunk0 + g * G, G)], idxb.at[gb],
                         isem)

    def idx_wait(gb):
        pltpu.make_async_copy(epairs.at[pl.ds(chunk0, G)], idxb.at[gb],
                              isem).wait()

    def start_gather(b, gb, j):
        pltpu.async_copy(h.at[idxb.at[gb, j, 0]], rows.at[b], gsems[b])

    def wait_gather(b):
        pltpu.make_async_copy(h.at[idxb.at[0, 0, 0]], rows.at[b],
                              gsems[b]).wait()

    def scatter(b, gb, j):
        pltpu.sync_copy(rows.at[b], acc.at[idxb.at[gb, j, 1]], add=True)

    pltpu.sync_copy(epairs.at[pl.ds(chunk0, G)], idxb.at[0])
    idx_start(1, 1)
    start_gather(0, 0, 0)

    def group(m, gb):
        for j in range(G):
            rb = j & 1
            wait_gather(rb)
            if j < G - 1:
                start_gather(rb ^ 1, gb, j + 1)
            else:
                @pl.when(m < ng - 1)
                def _():
                    idx_wait(gb ^ 1)
                    start_gather(rb ^ 1, gb ^ 1, 0)
            scatter(rb, gb, j)

        @pl.when(m < ng - 2)
        def _():
            idx_start(m + 2, gb)

    def body(mm, carry):
        group(2 * mm, 0)
        group(2 * mm + 1, 1)
        return carry

    lax.fori_loop(0, ng // 2, body, 0)


def _agg_scratch(F):
    return [
        pltpu.VMEM((2, G, 2, EC), jnp.int32),
        pltpu.VMEM((2, EC, F), jnp.float32),
        pltpu.VMEM_SHARED((HP, F), jnp.float32),
        pltpu.SemaphoreType.DMA,
        pltpu.SemaphoreType.DMA,
        pltpu.SemaphoreType.DMA,
    ]


def _make_agg(F):
    rpw = HP // NS
    nchunks = EP // EC // NS

    @functools.partial(
        pl.kernel,
        out_type=(jax.ShapeDtypeStruct((HP, F), jnp.float32),
                  jax.ShapeDtypeStruct((HP, F), jnp.float32)),
        mesh=_sc_mesh(),
        scratch_types=_agg_scratch(F),
    )
    def agg_kernel(h0, h1, zrows, epairs, o0, o1, idxb, rows, acc,
                   gsem0, gsem1, isem):
        c = lax.axis_index("c")
        s = lax.axis_index("s")
        r0 = s * rpw
        chunk0 = s * nchunks

        pltpu.sync_copy(zrows, acc.at[pl.ds(r0, rpw)])
        plsc.subcore_barrier()

        @pl.when(c == 0)
        def _():
            _edge_loop(h0, epairs, acc, idxb, rows, gsem0, gsem1, isem,
                       chunk0, nchunks)

        @pl.when(c == 1)
        def _():
            _edge_loop(h1, epairs, acc, idxb, rows, gsem0, gsem1, isem,
                       chunk0, nchunks)

        plsc.subcore_barrier()

        def writeback(o):
            pltpu.sync_copy(acc.at[pl.ds(r0, rpw)], o.at[pl.ds(r0, rpw)])

        @pl.when(c == 0)
        def _():
            writeback(o0)

        @pl.when(c == 1)
        def _():
            writeback(o1)

    return agg_kernel


def _make_agg_edge_split(F):
    rpw = HP // NS
    nchunks = EP // EC // NW

    @functools.partial(
        pl.kernel,
        out_type=(jax.ShapeDtypeStruct((HP, F), jnp.float32),
                  jax.ShapeDtypeStruct((HP, F), jnp.float32)),
        mesh=_sc_mesh(),
        scratch_types=_agg_scratch(F),
    )
    def agg_kernel(g, zrows, epairs, o0, o1, idxb, rows, acc,
                   gsem0, gsem1, isem):
        c = lax.axis_index("c")
        s = lax.axis_index("s")
        wid = c * NS + s
        r0 = s * rpw

        pltpu.sync_copy(zrows, acc.at[pl.ds(r0, rpw)])
        plsc.subcore_barrier()
        _edge_loop(g, epairs, acc, idxb, rows, gsem0, gsem1, isem,
                   wid * nchunks, nchunks)
        plsc.subcore_barrier()

        @pl.when(c == 0)
        def _():
            pltpu.sync_copy(acc.at[pl.ds(r0, rpw)], o0.at[pl.ds(r0, rpw)])

        @pl.when(c == 1)
        def _():
            pltpu.sync_copy(acc.at[pl.ds(r0, rpw)], o1.at[pl.ds(r0, rpw)])

    return agg_kernel


def _dinv_of(deg_blk):
    return lax.rsqrt(1.0 + jnp.sum(deg_blk, axis=0))


def _mm1_body(x_ref, w_ref, deg_ref, h0_ref, h1_ref):
    dinv = _dinv_of(deg_ref[...])
    h = jnp.dot(x_ref[...], w_ref[...], preferred_element_type=jnp.float32)
    h = h * dinv[:, None]
    h0_ref[...] = h[:, :HID_CH // 2]
    h1_ref[...] = h[:, HID_CH // 2:]


def _mm1(x_p, W1, deg_parts):
    grid = (HP // RB,)
    return pl.pallas_call(
        _mm1_body,
        grid=grid,
        in_specs=[
            pl.BlockSpec((RB, IN_CH), lambda i: (i, 0)),
            pl.BlockSpec((IN_CH, HID_CH), lambda i: (0, 0)),
            pl.BlockSpec((NW, RB), lambda i: (0, i)),
        ],
        out_specs=[
            pl.BlockSpec((RB, HID_CH // 2), lambda i: (i, 0)),
            pl.BlockSpec((RB, HID_CH // 2), lambda i: (i, 0)),
        ],
        out_shape=[
            jax.ShapeDtypeStruct((HP, HID_CH // 2), jnp.float32),
            jax.ShapeDtypeStruct((HP, HID_CH // 2), jnp.float32),
        ],
    )(x_p, W1, deg_parts)


def _mm2_body(a0_ref, a1_ref, h0_ref, h1_ref, deg_ref, b1_ref, w2_ref,
              g_ref):
    dinv = _dinv_of(deg_ref[...])
    b = b1_ref[...]
    t0 = a0_ref[...] + h0_ref[...]
    t1 = a1_ref[...] + h1_ref[...]
    z0 = jnp.maximum(t0 * dinv[:, None] + b[:, :HID_CH // 2], 0.0)
    z1 = jnp.maximum(t1 * dinv[:, None] + b[:, HID_CH // 2:], 0.0)
    w2 = w2_ref[...]
    h = jnp.dot(z0, w2[:HID_CH // 2], preferred_element_type=jnp.float32)
    h = h + jnp.dot(z1, w2[HID_CH // 2:], preferred_element_type=jnp.float32)
    g_ref[...] = h * dinv[:, None]


def _mm2(a0, a1, h0, h1, deg_parts, b1r, W2):
    grid = (HP // RB,)
    return pl.pallas_call(
        _mm2_body,
        grid=grid,
        in_specs=[
            pl.BlockSpec((RB, HID_CH // 2), lambda i: (i, 0)),
            pl.BlockSpec((RB, HID_CH // 2), lambda i: (i, 0)),
            pl.BlockSpec((RB, HID_CH // 2), lambda i: (i, 0)),
            pl.BlockSpec((RB, HID_CH // 2), lambda i: (i, 0)),
            pl.BlockSpec((NW, RB), lambda i: (0, i)),
            pl.BlockSpec((1, HID_CH), lambda i: (0, 0)),
            pl.BlockSpec((HID_CH, OUT_CH), lambda i: (0, 0)),
        ],
        out_specs=pl.BlockSpec((RB, OUT_CH), lambda i: (i, 0)),
        out_shape=jax.ShapeDtypeStruct((HP, OUT_CH), jnp.float32),
    )(a0, a1, h0, h1, deg_parts, b1r, W2)


def _mm3_body(c0_ref, c1_ref, g_ref2, deg_ref, b2_ref, out_ref):
    dinv = _dinv_of(deg_ref[...])
    o = c0_ref[...] + c1_ref[...] + g_ref2[...]
    out_ref[...] = o * dinv[:, None] + b2_ref[...]


def _mm3(c0, c1, g, deg_parts, b2r):
    grid = (HP // RB,)
    return pl.pallas_call(
        _mm3_body,
        grid=grid,
        in_specs=[
            pl.BlockSpec((RB, OUT_CH), lambda i: (i, 0)),
            pl.BlockSpec((RB, OUT_CH), lambda i: (i, 0)),
            pl.BlockSpec((RB, OUT_CH), lambda i: (i, 0)),
            pl.BlockSpec((NW, RB), lambda i: (0, i)),
            pl.BlockSpec((1, OUT_CH), lambda i: (0, 0)),
        ],
        out_specs=pl.BlockSpec((RB, OUT_CH), lambda i: (i, 0)),
        out_shape=jax.ShapeDtypeStruct((N_NODES, OUT_CH), jnp.float32),
    )(c0, c1, g, deg_parts, b2r)


_deg_kernel = _make_deg()
_agg_hid = _make_agg(HID_CH // 2)
_agg_out = _make_agg_edge_split(OUT_CH)


def kernel(x, edge_index, W1, b1, W2, b2):
    src = edge_index[0].astype(jnp.int32)
    dst = edge_index[1].astype(jnp.int32)
    pad_e = EP - N_EDGES
    spare = HP - N_NODES
    pad_iota = jnp.arange(pad_e, dtype=jnp.int32)
    pad_dst = DUMP + (pad_iota % spare)
    pad_src = DUMP + ((pad_iota + spare // 2) % spare)
    src_p = jnp.concatenate([src, pad_src])
    dst_p = jnp.concatenate([dst, pad_dst])
    epairs = jnp.stack([src_p.reshape(EP // EC, EC),
                        dst_p.reshape(EP // EC, EC)], axis=1)
    zrows = jnp.zeros((HP // NS, OUT_CH), jnp.float32)

    deg_parts = _deg_kernel(dst_p)
    h0, h1 = _mm1(x, W1, deg_parts)
    a0, a1 = _agg_hid(h0, h1, zrows, epairs)
    g = _mm2(a0, a1, h0, h1, deg_parts,
             b1.reshape(1, HID_CH), W2)
    c0, c1 = _agg_out(g, zrows, epairs)
    return _mm3(c0, c1, g, deg_parts,
                b2.reshape(1, OUT_CH))
